# baseline (speedup 1.0000x reference)
import functools

import jax
import jax.numpy as jnp
from jax import lax
from jax.experimental import pallas as pl
from jax.experimental.pallas import tpu as pltpu
from jax.experimental.pallas import tpu_sc as plsc

MEM_N = 16777216
I32 = jnp.int32
F32 = jnp.float32
_M16 = 0xFFFF
_M255 = 255


def _b2i(c):
    return jnp.where(c, jnp.int32(1), jnp.int32(0))


def _b2f(c):
    return jnp.where(c, F32(1.0), F32(0.0))


def _srl(x, n):
    return lax.shift_right_logical(x, jnp.full_like(x, n))


def _u32f(x):
    lo31 = (x & jnp.int32(0x7FFFFFFF)).astype(F32)
    return lo31 + jnp.where(x < 0, F32(2147483648.0), F32(0.0))


def _pair_f32(lo, hi):
    neg = hi < 0
    ml = jnp.where(neg, -lo, lo)
    mh = jnp.where(neg, ~hi + _b2i(lo == 0), hi)
    vf = _u32f(mh) * F32(4294967296.0) + _u32f(ml)
    return jnp.where(neg, -vf, vf)


def _trunc_rt(v):
    small = jnp.abs(v) < F32(8388608.0)
    return jnp.where(small, v.astype(I32).astype(F32), v)


def _rbf16(x):
    c = x * F32(65537.0)
    return c - (c - x)


def _mul64(sl, sh, ax):
    s0 = sl & _M16
    s1 = _srl(sl, 16)
    s2 = sh & _M16
    s3 = _srl(sh, 16)
    a0 = ax & _M16
    a1 = _srl(ax, 16)
    t0 = s0 * a0
    r0 = t0 & _M16
    c1 = _srl(t0, 16)
    t1a = s1 * a0
    t1b = s0 * a1
    sum1 = (t1a & _M16) + (t1b & _M16) + c1
    r1 = sum1 & _M16
    c2 = _srl(sum1, 16) + _srl(t1a, 16) + _srl(t1b, 16)
    t2a = s2 * a0
    t2b = s1 * a1
    sum2 = (t2a & _M16) + (t2b & _M16) + c2
    r2 = sum2 & _M16
    c3 = _srl(sum2, 16) + _srl(t2a, 16) + _srl(t2b, 16)
    t3a = s3 * a0
    t3b = s2 * a1
    sum3 = (t3a & _M16) + (t3b & _M16) + c3
    r3 = sum3 & _M16
    return r0 | (r1 << 16), r2 | (r3 << 16)


def _divmod64(sl, sh, d):
    neg = sh < 0
    ul = jnp.where(neg, -sl, sl)
    uh = jnp.where(neg, ~sh + _b2i(sl == 0), sh)
    rcp = F32(1.0) / d.astype(F32)
    r = sl - sl
    qlo = r
    qhi = r
    digits = [
        _srl(uh, 24) & _M255, _srl(uh, 16) & _M255, _srl(uh, 8) & _M255, uh & _M255,
        _srl(ul, 24) & _M255, _srl(ul, 16) & _M255, _srl(ul, 8) & _M255, ul & _M255,
    ]
    for b in digits:
        x = (r << 8) | b
        q = (_u32f(x) * rcp).astype(I32)
        rem = x - q * d
        for _ in range(2):
            q = jnp.where(rem < 0, q - 1, q)
            rem = jnp.where(rem < 0, rem + d, rem)
        for _ in range(2):
            q = jnp.where(rem >= d, q + 1, q)
            rem = jnp.where(rem >= d, rem - d, rem)
        qhi = (qhi << 8) | _srl(qlo, 24)
        qlo = (qlo << 8) | q
        r = rem
    rnz = _b2i(r != 0)
    inc_lo = qlo + rnz
    carry = _b2i((inc_lo == 0) & (rnz == 1))
    inc_hi = qhi + carry
    nql = -inc_lo
    nqh = ~inc_hi + _b2i(inc_lo == 0)
    out_qlo = jnp.where(neg, nql, qlo)
    out_qhi = jnp.where(neg, nqh, qhi)
    out_r = jnp.where(neg, jnp.where(rnz == 1, d - r, r), r)
    return out_qlo, out_qhi, out_r


def _ucmp_lt(a, b):
    bias = jnp.int32(-2147483648)
    return (a ^ bias) < (b ^ bias)


def _body(mem_hbm, regs_hbm, w1_hbm, w2_hbm, b1_hbm, b2_hbm, out_hbm,
          regs_v, b0v, b1v_, b2v_, b3v, b4v, b5v, b6v, b7v,
          w1v, bb1v, wcolv, b2cv, outv,
          sem_w, sem_g, sem_c):
    is_lead = (lax.axis_index("c") == 0) & (lax.axis_index("s") == 0)

    @pl.when(is_lead)
    def _():
        lane = lax.iota(I32, 16)
        zf = jnp.zeros((16,), F32)

        cw = [
            pltpu.make_async_copy(w1_hbm, w1v, sem_w),
            pltpu.make_async_copy(b1_hbm, bb1v, sem_w),
        ]
        for c in cw:
            c.start()

        pltpu.sync_copy(regs_hbm, regs_v)
        rv = regs_v[...]
        pc_s = rv[0]
        sp_s = rv[1]
        bp_s = rv[2]
        ax_s = rv[3]

        base = jnp.where(
            lane == 0, jnp.full((16,), pc_s, I32),
            jnp.where(lane == 1, jnp.full((16,), sp_s, I32),
                      jnp.where(lane == 2, jnp.full((16,), ax_s, I32),
                                jnp.where(lane == 3, jnp.full((16,), bp_s, I32),
                                          jnp.where(lane == 4,
                                                    jnp.full((16,), bp_s + 8, I32),
                                                    jnp.zeros((16,), I32))))))
        gbufs = [b0v, b1v_, b2v_, b3v, b4v, b5v, b6v, b7v]
        cg = []
        for k, gb in enumerate(gbufs):
            c = pltpu.make_async_copy(mem_hbm.at[base + k], gb, sem_g)
            c.start()
            cg.append(c)
        for c in cg:
            c.wait()

        bv = [plsc.bitcast(gb[...], I32) for gb in gbufs]
        lo_vec = bv[0] + (bv[1] << 8) + (bv[2] << 16) + (bv[3] << 24)
        hi_vec = bv[4] + (bv[5] << 8) + (bv[6] << 16) + (bv[7] << 24)

        i_lo = lo_vec[0]
        i_hi = hi_vec[0]
        sl = lo_vec[1]
        sh = hi_vec[1]
        opcode = i_lo & 255
        imm_lo = _srl(i_lo, 8) | (i_hi << 24)
        imm_hi = i_hi >> 8

        oc = jnp.minimum(opcode, jnp.int32(38))
        cb = oc * 4
        cc = []
        for j in range(4):
            for c in range(4):
                idx = lane * 156 + (c * 2496 + cb + j)
                h = pltpu.make_async_copy(
                    w2_hbm.at[idx], wcolv.at[pl.ds((j * 4 + c) * 16, 16)], sem_c)
                h.start()
                cc.append(h)
        hb2 = pltpu.make_async_copy(b2_hbm.at[cb + (lane & 3)], b2cv, sem_c)
        hb2.start()
        cc.append(hb2)

        xs = [
            _rbf16(pc_s.astype(F32)), _rbf16(sp_s.astype(F32)),
            _rbf16(bp_s.astype(F32)), _rbf16(ax_s.astype(F32)),
            _rbf16(_pair_f32(imm_lo, imm_hi)), _rbf16(_pair_f32(sl, sh)),
            _rbf16(_pair_f32(lo_vec[2], hi_vec[2])),
            _rbf16(_pair_f32(lo_vec[3], hi_vec[3])),
            _rbf16(_pair_f32(lo_vec[4], hi_vec[4])),
        ]

        for c in cw:
            c.wait()

        accs = []
        for c in range(4):
            acc = zf
            for k in range(9):
                acc = acc + xs[k] * w1v[pl.ds(k * 64 + c * 16, 16)]
            accs.append(_rbf16(acc + bb1v[pl.ds(c * 16, 16)]))

        for h in cc:
            h.wait()

        b2c = b2cv[...]
        sel = []
        for j in range(4):
            p = zf
            for c in range(4):
                p = p + accs[c] * wcolv[pl.ds((j * 4 + c) * 16, 16)]
            tot = F32(0.0)
            for l in range(16):
                tot = tot + p[l]
            tot = tot + b2c[j]
            sel.append(_trunc_rt(_rbf16(tot)))

        new_pc_f, new_sp_f, new_bp_f, new_ax_f = sel

        is_mul = _b2f(opcode == 16)
        is_div = _b2f(opcode == 17)
        is_mod = _b2f(opcode == 18)
        plo, phi = _mul64(sl, sh, ax_s)
        mul_f = _pair_f32(plo, phi)
        axv = jnp.full((16,), ax_s, I32)
        dv = jnp.where(axv == 0, jnp.full((16,), 1, I32), axv)
        qlo_v, qhi_v, rem_v = _divmod64(
            jnp.full((16,), sl, I32), jnp.full((16,), sh, I32), dv)
        div_f = _pair_f32(qlo_v, qhi_v)
        mod_f = rem_v.astype(F32)
        na = (
            new_ax_f * (F32(1.0) - is_mul - is_div - is_mod)
            + mul_f * is_mul + div_f * is_div + mod_f * is_mod
        )
        na = _trunc_rt(na)

        is_eq = _b2f(opcode == 24)
        is_ne = _b2f(opcode == 25)
        is_lt = _b2f(opcode == 26)
        is_gt = _b2f(opcode == 27)
        is_le = _b2f(opcode == 28)
        is_ge = _b2f(opcode == 29)
        eqb = (sh == 0) & (sl == ax_s)
        ltb = (sh < 0) | ((sh == 0) & _ucmp_lt(sl, ax_s))
        na = (
            na * (F32(1.0) - is_eq - is_ne - is_lt - is_gt - is_le - is_ge)
            + _b2f(eqb) * is_eq
            + _b2f(~eqb) * is_ne
            + _b2f(ltb) * is_lt
            + _b2f(~ltb & ~eqb) * is_gt
            + _b2f(ltb | eqb) * is_le
            + _b2f(~ltb) * is_ge
        )

        res = jnp.where(
            lane == 0, jnp.full((16,), new_pc_f, F32),
            jnp.where(lane == 1, jnp.full((16,), new_sp_f, F32),
                      jnp.where(lane == 2, jnp.full((16,), new_bp_f, F32),
                                jnp.where(lane == 3, na, zf))))
        outv[...] = res
        pltpu.sync_copy(outv, out_hbm)


def kernel(pc, sp, bp, ax, memory, W1, b1, W2, b2):
    mem32 = memory.astype(jnp.uint32)
    regs = jnp.concatenate([
        jnp.stack([pc, sp, bp, ax]).astype(jnp.uint32).astype(I32),
        jnp.zeros((12,), I32),
    ])
    mesh = plsc.VectorSubcoreMesh(core_axis_name="c", subcore_axis_name="s")
    run = functools.partial(
        pl.kernel,
        mesh=mesh,
        out_type=jax.ShapeDtypeStruct((16,), F32),
        scratch_types=[
            pltpu.VMEM((16,), I32),
            pltpu.VMEM((16,), jnp.uint32),
            pltpu.VMEM((16,), jnp.uint32),
            pltpu.VMEM((16,), jnp.uint32),
            pltpu.VMEM((16,), jnp.uint32),
            pltpu.VMEM((16,), jnp.uint32),
            pltpu.VMEM((16,), jnp.uint32),
            pltpu.VMEM((16,), jnp.uint32),
            pltpu.VMEM((16,), jnp.uint32),
            pltpu.VMEM((576,), F32),
            pltpu.VMEM((64,), F32),
            pltpu.VMEM((256,), F32),
            pltpu.VMEM((16,), F32),
            pltpu.VMEM((16,), F32),
            pltpu.SemaphoreType.DMA,
            pltpu.SemaphoreType.DMA,
            pltpu.SemaphoreType.DMA,
        ],
    )(_body)
    out = run(
        mem32, regs,
        W1.astype(F32).reshape(576),
        W2.astype(F32).reshape(9984),
        b1.astype(F32),
        b2.astype(F32),
    )
    o64 = out[:4].astype(jnp.int64)
    return o64[0], o64[1], o64[2], o64[3]


# speedup vs baseline: 1.0010x; 1.0010x over previous
import functools

import jax
import jax.numpy as jnp
from jax import lax
from jax.experimental import pallas as pl
from jax.experimental.pallas import tpu as pltpu
from jax.experimental.pallas import tpu_sc as plsc

I32 = jnp.int32
F32 = jnp.float32
_M16 = 0xFFFF
_M255 = 255


def _b2i(c):
    return jnp.where(c, jnp.int32(1), jnp.int32(0))


def _b2f(c):
    return jnp.where(c, F32(1.0), F32(0.0))


def _srl(x, n):
    return lax.shift_right_logical(x, jnp.full_like(x, n))


def _u32f(x):
    lo31 = (x & jnp.int32(0x7FFFFFFF)).astype(F32)
    return lo31 + jnp.where(x < 0, F32(2147483648.0), F32(0.0))


def _pair_f32(lo, hi):
    neg = hi < 0
    ml = jnp.where(neg, -lo, lo)
    mh = jnp.where(neg, ~hi + _b2i(lo == 0), hi)
    vf = _u32f(mh) * F32(4294967296.0) + _u32f(ml)
    return jnp.where(neg, -vf, vf)


def _trunc_rt(v):
    small = jnp.abs(v) < F32(8388608.0)
    return jnp.where(small, v.astype(I32).astype(F32), v)


def _rbf16(x):
    c = x * F32(65537.0)
    return c - (c - x)


def _mul64(sl, sh, ax):
    s0 = sl & _M16
    s1 = _srl(sl, 16)
    s2 = sh & _M16
    s3 = _srl(sh, 16)
    a0 = ax & _M16
    a1 = _srl(ax, 16)
    t0 = s0 * a0
    r0 = t0 & _M16
    c1 = _srl(t0, 16)
    t1a = s1 * a0
    t1b = s0 * a1
    sum1 = (t1a & _M16) + (t1b & _M16) + c1
    r1 = sum1 & _M16
    c2 = _srl(sum1, 16) + _srl(t1a, 16) + _srl(t1b, 16)
    t2a = s2 * a0
    t2b = s1 * a1
    sum2 = (t2a & _M16) + (t2b & _M16) + c2
    r2 = sum2 & _M16
    c3 = _srl(sum2, 16) + _srl(t2a, 16) + _srl(t2b, 16)
    t3a = s3 * a0
    t3b = s2 * a1
    sum3 = (t3a & _M16) + (t3b & _M16) + c3
    r3 = sum3 & _M16
    return r0 | (r1 << 16), r2 | (r3 << 16)


def _divmod64(sl, sh, d):
    neg = sh < 0
    ul = jnp.where(neg, -sl, sl)
    uh = jnp.where(neg, ~sh + _b2i(sl == 0), sh)
    rcp = F32(1.0) / d.astype(F32)
    r = sl - sl
    qlo = r
    qhi = r
    digits = [
        _srl(uh, 24) & _M255, _srl(uh, 16) & _M255, _srl(uh, 8) & _M255, uh & _M255,
        _srl(ul, 24) & _M255, _srl(ul, 16) & _M255, _srl(ul, 8) & _M255, ul & _M255,
    ]
    for b in digits:
        x = (r << 8) | b
        q = (_u32f(x) * rcp).astype(I32)
        rem = x - q * d
        for _ in range(2):
            q = jnp.where(rem < 0, q - 1, q)
            rem = jnp.where(rem < 0, rem + d, rem)
        for _ in range(2):
            q = jnp.where(rem >= d, q + 1, q)
            rem = jnp.where(rem >= d, rem - d, rem)
        qhi = (qhi << 8) | _srl(qlo, 24)
        qlo = (qlo << 8) | q
        r = rem
    rnz = _b2i(r != 0)
    inc_lo = qlo + rnz
    carry = _b2i((inc_lo == 0) & (rnz == 1))
    inc_hi = qhi + carry
    nql = -inc_lo
    nqh = ~inc_hi + _b2i(inc_lo == 0)
    out_qlo = jnp.where(neg, nql, qlo)
    out_qhi = jnp.where(neg, nqh, qhi)
    out_r = jnp.where(neg, jnp.where(rnz == 1, d - r, r), r)
    return out_qlo, out_qhi, out_r


def _ucmp_lt(a, b):
    bias = jnp.int32(-2147483648)
    return (a ^ bias) < (b ^ bias)


def _body(mem_hbm, regs_hbm, w1_hbm, w2_hbm, b1_hbm, b2_hbm, out_hbm,
          regs_v, b0v, b1v_, b2v_, b3v, b4v, b5v, b6v, b7v,
          w1v, bb1v, wcolv, b2cv, outv,
          sem_w, sem_g, sem_c):
    is_lead = (lax.axis_index("c") == 0) & (lax.axis_index("s") == 0)

    @pl.when(is_lead)
    def _():
        lane = lax.iota(I32, 16)
        zf = jnp.zeros((16,), F32)

        cw = [
            pltpu.make_async_copy(w1_hbm, w1v, sem_w),
            pltpu.make_async_copy(b1_hbm, bb1v, sem_w),
        ]
        for c in cw:
            c.start()

        pltpu.sync_copy(regs_hbm, regs_v)
        rv = regs_v[...]
        pc_s = rv[0]
        sp_s = rv[1]
        bp_s = rv[2]
        ax_s = rv[3]

        base = jnp.where(
            lane == 0, jnp.full((16,), pc_s, I32),
            jnp.where(lane == 1, jnp.full((16,), sp_s, I32),
                      jnp.where(lane == 2, jnp.full((16,), ax_s, I32),
                                jnp.where(lane == 3, jnp.full((16,), bp_s, I32),
                                          jnp.where(lane == 4,
                                                    jnp.full((16,), bp_s + 8, I32),
                                                    jnp.zeros((16,), I32))))))
        gbufs = [b0v, b1v_, b2v_, b3v, b4v, b5v, b6v, b7v]
        cg = []
        for k, gb in enumerate(gbufs):
            c = pltpu.make_async_copy(mem_hbm.at[base + k], gb, sem_g)
            c.start()
            cg.append(c)
        for c in cg:
            c.wait()

        bv = [plsc.bitcast(gb[...], I32) for gb in gbufs]
        lo_vec = bv[0] + (bv[1] << 8) + (bv[2] << 16) + (bv[3] << 24)
        hi_vec = bv[4] + (bv[5] << 8) + (bv[6] << 16) + (bv[7] << 24)

        i_lo = lo_vec[0]
        i_hi = hi_vec[0]
        sl = lo_vec[1]
        sh = hi_vec[1]
        opcode = i_lo & 255
        imm_lo = _srl(i_lo, 8) | (i_hi << 24)
        imm_hi = i_hi >> 8

        oc = jnp.minimum(opcode, jnp.int32(38))
        cb = oc * 4
        cc = []
        for j in range(4):
            for c in range(4):
                idx = lane * 156 + (c * 2496 + cb + j)
                h = pltpu.make_async_copy(
                    w2_hbm.at[idx], wcolv.at[pl.ds((j * 4 + c) * 16, 16)], sem_c)
                h.start()
                cc.append(h)
        hb2 = pltpu.make_async_copy(b2_hbm.at[cb + (lane & 3)], b2cv, sem_c)
        hb2.start()
        cc.append(hb2)

        xs = [
            _rbf16(pc_s.astype(F32)), _rbf16(sp_s.astype(F32)),
            _rbf16(bp_s.astype(F32)), _rbf16(ax_s.astype(F32)),
            _rbf16(_pair_f32(imm_lo, imm_hi)), _rbf16(_pair_f32(sl, sh)),
            _rbf16(_pair_f32(lo_vec[2], hi_vec[2])),
            _rbf16(_pair_f32(lo_vec[3], hi_vec[3])),
            _rbf16(_pair_f32(lo_vec[4], hi_vec[4])),
        ]

        for c in cw:
            c.wait()

        accs = []
        for c in range(4):
            acc = zf
            for k in range(9):
                acc = acc + xs[k] * w1v[pl.ds(k * 64 + c * 16, 16)]
            accs.append(_rbf16(acc + bb1v[pl.ds(c * 16, 16)]))

        for h in cc:
            h.wait()

        b2c = b2cv[...]
        sel = []
        for j in range(4):
            p = zf
            for c in range(4):
                p = p + accs[c] * wcolv[pl.ds((j * 4 + c) * 16, 16)]
            tot = F32(0.0)
            for l in range(16):
                tot = tot + p[l]
            tot = tot + b2c[j]
            sel.append(_trunc_rt(_rbf16(tot)))

        new_pc_f, new_sp_f, new_bp_f, new_ax_f = sel

        is_mul = _b2f(opcode == 16)
        is_div = _b2f(opcode == 17)
        is_mod = _b2f(opcode == 18)
        plo, phi = _mul64(sl, sh, ax_s)
        mul_f = _pair_f32(plo, phi)
        axv = jnp.full((16,), ax_s, I32)
        dv = jnp.where(axv == 0, jnp.full((16,), 1, I32), axv)
        qlo_v, qhi_v, rem_v = _divmod64(
            jnp.full((16,), sl, I32), jnp.full((16,), sh, I32), dv)
        div_f = _pair_f32(qlo_v, qhi_v)
        mod_f = rem_v.astype(F32)
        na = (
            new_ax_f * (F32(1.0) - is_mul - is_div - is_mod)
            + mul_f * is_mul + div_f * is_div + mod_f * is_mod
        )
        na = _trunc_rt(na)

        is_eq = _b2f(opcode == 24)
        is_ne = _b2f(opcode == 25)
        is_lt = _b2f(opcode == 26)
        is_gt = _b2f(opcode == 27)
        is_le = _b2f(opcode == 28)
        is_ge = _b2f(opcode == 29)
        eqb = (sh == 0) & (sl == ax_s)
        ltb = (sh < 0) | ((sh == 0) & _ucmp_lt(sl, ax_s))
        na = (
            na * (F32(1.0) - is_eq - is_ne - is_lt - is_gt - is_le - is_ge)
            + _b2f(eqb) * is_eq
            + _b2f(~eqb) * is_ne
            + _b2f(ltb) * is_lt
            + _b2f(~ltb & ~eqb) * is_gt
            + _b2f(ltb | eqb) * is_le
            + _b2f(~ltb) * is_ge
        )

        res = jnp.where(
            lane == 0, jnp.full((16,), new_pc_f, F32),
            jnp.where(lane == 1, jnp.full((16,), new_sp_f, F32),
                      jnp.where(lane == 2, jnp.full((16,), new_bp_f, F32),
                                jnp.where(lane == 3, na, zf))))
        outv[...] = res
        pltpu.sync_copy(outv, out_hbm)


def kernel(pc, sp, bp, ax, memory, W1, b1, W2, b2):
    mem32 = memory.astype(jnp.uint32)
    regs = jnp.concatenate([
        jnp.stack([pc, sp, bp, ax]).astype(jnp.uint32).astype(I32),
        jnp.zeros((12,), I32),
    ])
    mesh = plsc.VectorSubcoreMesh(core_axis_name="c", subcore_axis_name="s")
    run = functools.partial(
        pl.kernel,
        mesh=mesh,
        out_type=jax.ShapeDtypeStruct((16,), F32),
        scratch_types=[
            pltpu.VMEM((16,), I32),
            pltpu.VMEM((16,), jnp.uint32),
            pltpu.VMEM((16,), jnp.uint32),
            pltpu.VMEM((16,), jnp.uint32),
            pltpu.VMEM((16,), jnp.uint32),
            pltpu.VMEM((16,), jnp.uint32),
            pltpu.VMEM((16,), jnp.uint32),
            pltpu.VMEM((16,), jnp.uint32),
            pltpu.VMEM((16,), jnp.uint32),
            pltpu.VMEM((576,), F32),
            pltpu.VMEM((64,), F32),
            pltpu.VMEM((256,), F32),
            pltpu.VMEM((16,), F32),
            pltpu.VMEM((16,), F32),
            pltpu.SemaphoreType.DMA,
            pltpu.SemaphoreType.DMA,
            pltpu.SemaphoreType.DMA,
        ],
    )(_body)
    out = run(
        mem32, regs,
        W1.astype(F32).reshape(576),
        W2.astype(F32).reshape(9984),
        b1.astype(F32),
        b2.astype(F32),
    )
    o64 = out[:4].astype(jnp.int64)
    return o64[0], o64[1], o64[2], o64[3]
